# Initial kernel scaffold; baseline (speedup 1.0000x reference)
import jax
import jax.numpy as jnp
from jax.experimental import pallas as pl


def kernel(z, embed_weight):
    raise NotImplementedError("write your pallas kernel here")


# speedup vs baseline: 1.0467x; 1.0467x over previous
import functools

import jax
import jax.numpy as jnp
from jax import lax
from jax.experimental import pallas as pl
from jax.experimental.pallas import tpu as pltpu
from jax.experimental.pallas import tpu_sc as plsc

NUM_HIDDENS = 768
N_EMBED = 8192

TR = 512
TCB = 2048
R = None
C = N_EMBED // TCB


def _argmin_body(f_ref, e_ref, ind_ref, sum_ref, best_val, best_idx, e2_ref):
    r = pl.program_id(0)
    c = pl.program_id(1)
    num_c = pl.num_programs(1)

    f = f_ref[...]
    e = e_ref[...]

    @pl.when(r == 0)
    def _():
        e2_ref[:, pl.ds(c * TCB, TCB)] = jnp.sum(e * e, axis=1)[None, :]

    e2 = e2_ref[0, pl.ds(c * TCB, TCB)][None, :]
    sumf = jnp.sum(f * f, axis=1, keepdims=True)

    scores = lax.dot_general(f, e, (((1,), (1,)), ((), ())),
                             preferred_element_type=jnp.float32)
    neg = -((sumf - 2.0 * scores) + e2)

    maxv = jnp.max(neg, axis=1, keepdims=True)
    iota = lax.broadcasted_iota(jnp.int32, (TR, TCB), 1)
    cand = jnp.where(neg == maxv, iota, TCB)
    local_idx = jnp.min(cand, axis=1, keepdims=True)
    gidx = c * TCB + local_idx

    @pl.when(c == 0)
    def _():
        best_val[...] = maxv
        best_idx[...] = gidx

    @pl.when(c > 0)
    def _():
        upd = maxv > best_val[...]
        best_val[...] = jnp.where(upd, maxv, best_val[...])
        best_idx[...] = jnp.where(upd, gidx, best_idx[...])

    @pl.when(c == num_c - 1)
    def _():
        ind_ref[0, 0, :] = best_idx[:, 0]
        tile_sum = -jnp.sum(best_val[...])[None, None]
        sum_ref[...] = jnp.where(r == 0, tile_sum, sum_ref[...] + tile_sum)


def _make_sc_gather(n_rows, d, table_rows):
    info = plsc.get_sparse_core_info()
    nw = info.num_cores * info.num_subcores
    assert n_rows % nw == 0
    b_per_w = n_rows // nw
    mesh = plsc.VectorSubcoreMesh(core_axis_name="c", subcore_axis_name="s")

    @functools.partial(
        pl.kernel,
        mesh=mesh,
        out_type=jax.ShapeDtypeStruct((n_rows, d), jnp.float32),
        scratch_types=[
            pltpu.VMEM((b_per_w,), jnp.int32),
            pltpu.VMEM((b_per_w, d), jnp.float32),
            pltpu.SemaphoreType.DMA,
        ],
    )
    def gather_k(table_hbm, idx_hbm, out_hbm, idx_v, rows_v, sem):
        wid = lax.axis_index("s") * info.num_cores + lax.axis_index("c")
        base = wid * b_per_w
        pltpu.sync_copy(idx_hbm.at[pl.ds(base, b_per_w)], idx_v)
        pltpu.async_copy(table_hbm.at[idx_v], rows_v, sem).wait()
        pltpu.sync_copy(rows_v, out_hbm.at[pl.ds(base, b_per_w)])

    return gather_k


def kernel(z, embed_weight):
    B, H, ch = z.shape
    n = B * H
    global R
    R = n // TR

    flatten = z.reshape(n, ch)

    ind3, dist_sum = pl.pallas_call(
        _argmin_body,
        grid=(n // TR, C),
        in_specs=[
            pl.BlockSpec((TR, ch), lambda r, c: (r, 0)),
            pl.BlockSpec((TCB, ch), lambda r, c: (c, 0)),
        ],
        out_specs=[
            pl.BlockSpec((1, 1, TR), lambda r, c: (r, 0, 0)),
            pl.BlockSpec((1, 1), lambda r, c: (0, 0)),
        ],
        out_shape=[
            jax.ShapeDtypeStruct((n // TR, 1, TR), jnp.int32),
            jax.ShapeDtypeStruct((1, 1), jnp.float32),
        ],
        scratch_shapes=[
            pltpu.VMEM((TR, 1), jnp.float32),
            pltpu.VMEM((TR, 1), jnp.int32),
            pltpu.VMEM((1, N_EMBED), jnp.float32),
        ],
    )(flatten, embed_weight)

    ind_flat = ind3.reshape(n)
    gather = _make_sc_gather(n, ch, embed_weight.shape[0])
    z_q = gather(embed_weight, ind_flat).reshape(B, H, ch)

    diff = dist_sum[0, 0] * (12.5 / (n * ch))
    ind = ind_flat.reshape(B, H)
    return z_q, diff, ind


# speedup vs baseline: 1.2594x; 1.2033x over previous
import functools

import jax
import jax.numpy as jnp
from jax import lax
from jax.experimental import pallas as pl
from jax.experimental.pallas import tpu as pltpu
from jax.experimental.pallas import tpu_sc as plsc

NUM_HIDDENS = 768
N_EMBED = 8192

TR = 512
TCB = 2048
R = None
C = N_EMBED // TCB


UNROLL = 4
TCH = TCB // UNROLL


def _argmin_body(f_ref, e_ref, ind_ref, sum_ref, best_val, best_idx, e2_ref):
    r = pl.program_id(0)
    c = pl.program_id(1)
    num_c = pl.num_programs(1)

    f = f_ref[...]

    @pl.when(r == 0)
    def _():
        e = e_ref[...]
        e2_ref[:, pl.ds(c * TCB, TCB)] = jnp.sum(e * e, axis=1)[None, :]

    sumf = jnp.sum(f * f, axis=1, keepdims=True)
    iota = lax.broadcasted_iota(jnp.int32, (TR, TCH), 1)

    run_val = None
    run_idx = None
    for j in range(UNROLL):
        e_j = e_ref[pl.ds(j * TCH, TCH), :]
        s = lax.dot_general(f, e_j, (((1,), (1,)), ((), ())),
                            preferred_element_type=jnp.float32)
        e2_j = e2_ref[0, pl.ds(c * TCB + j * TCH, TCH)][None, :]
        dist = (sumf - 2.0 * s) + e2_j
        minv = jnp.min(dist, axis=1, keepdims=True)
        cand = jnp.where(dist == minv, iota, TCH)
        lidx = jnp.min(cand, axis=1, keepdims=True) + (c * TCB + j * TCH)
        if j == 0:
            run_val, run_idx = minv, lidx
        else:
            upd = minv < run_val
            run_val = jnp.where(upd, minv, run_val)
            run_idx = jnp.where(upd, lidx, run_idx)

    @pl.when(c == 0)
    def _():
        best_val[...] = run_val
        best_idx[...] = run_idx

    @pl.when(c > 0)
    def _():
        upd = run_val < best_val[...]
        best_val[...] = jnp.where(upd, run_val, best_val[...])
        best_idx[...] = jnp.where(upd, run_idx, best_idx[...])

    @pl.when(c == num_c - 1)
    def _():
        ind_ref[0, 0, :] = best_idx[:, 0]
        tile_sum = jnp.sum(best_val[...])[None, None]
        sum_ref[...] = jnp.where(r == 0, tile_sum, sum_ref[...] + tile_sum)


def _make_sc_gather(n_rows, d, table_rows):
    info = plsc.get_sparse_core_info()
    nw = info.num_cores * info.num_subcores
    assert n_rows % nw == 0
    b_per_w = n_rows // nw
    mesh = plsc.VectorSubcoreMesh(core_axis_name="c", subcore_axis_name="s")

    @functools.partial(
        pl.kernel,
        mesh=mesh,
        out_type=jax.ShapeDtypeStruct((n_rows, d), jnp.float32),
        scratch_types=[
            pltpu.VMEM((b_per_w,), jnp.int32),
            pltpu.VMEM((b_per_w, d), jnp.float32),
            pltpu.SemaphoreType.DMA,
        ],
    )
    def gather_k(table_hbm, idx_hbm, out_hbm, idx_v, rows_v, sem):
        wid = lax.axis_index("s") * info.num_cores + lax.axis_index("c")
        base = wid * b_per_w
        pltpu.sync_copy(idx_hbm.at[pl.ds(base, b_per_w)], idx_v)
        pltpu.async_copy(table_hbm.at[idx_v], rows_v, sem).wait()
        pltpu.sync_copy(rows_v, out_hbm.at[pl.ds(base, b_per_w)])

    return gather_k


def kernel(z, embed_weight):
    B, H, ch = z.shape
    n = B * H
    global R
    R = n // TR

    flatten = z.reshape(n, ch)

    ind3, dist_sum = pl.pallas_call(
        _argmin_body,
        grid=(n // TR, C),
        in_specs=[
            pl.BlockSpec((TR, ch), lambda r, c: (r, 0)),
            pl.BlockSpec((TCB, ch), lambda r, c: (c, 0)),
        ],
        out_specs=[
            pl.BlockSpec((1, 1, TR), lambda r, c: (r, 0, 0)),
            pl.BlockSpec((1, 1), lambda r, c: (0, 0)),
        ],
        out_shape=[
            jax.ShapeDtypeStruct((n // TR, 1, TR), jnp.int32),
            jax.ShapeDtypeStruct((1, 1), jnp.float32),
        ],
        scratch_shapes=[
            pltpu.VMEM((TR, 1), jnp.float32),
            pltpu.VMEM((TR, 1), jnp.int32),
            pltpu.VMEM((1, N_EMBED), jnp.float32),
        ],
    )(flatten, embed_weight)

    ind_flat = ind3.reshape(n)
    gather = _make_sc_gather(n, ch, embed_weight.shape[0])
    z_q = gather(embed_weight, ind_flat).reshape(B, H, ch)

    diff = dist_sum[0, 0] * (12.5 / (n * ch))
    ind = ind_flat.reshape(B, H)
    return z_q, diff, ind


# speedup vs baseline: 1.2803x; 1.0165x over previous
import functools

import jax
import jax.numpy as jnp
from jax import lax
from jax.experimental import pallas as pl
from jax.experimental.pallas import tpu as pltpu
from jax.experimental.pallas import tpu_sc as plsc

NUM_HIDDENS = 768
N_EMBED = 8192

TR = 512
TCB = 2048
C = N_EMBED // TCB
UNROLL = 4
TCH = TCB // UNROLL


def _argmin_body(f_ref, e_ref, ind_ref, sum_ref, bv_ref, bi_ref, e2_ref):
    c = pl.program_id(0)
    r = pl.program_id(1)
    num_c = pl.num_programs(0)

    f = f_ref[...]

    @pl.when(r == 0)
    def _():
        e = e_ref[...]
        e2_ref[...] = jnp.sum(e * e, axis=1)[None, :]

    sumf = jnp.sum(f * f, axis=1, keepdims=True)
    iota = lax.broadcasted_iota(jnp.int32, (TR, TCH), 1)

    run_val = None
    run_idx = None
    for j in range(UNROLL):
        e_j = e_ref[pl.ds(j * TCH, TCH), :]
        s = lax.dot_general(f, e_j, (((1,), (1,)), ((), ())),
                            preferred_element_type=jnp.float32)
        e2_j = e2_ref[0, pl.ds(j * TCH, TCH)][None, :]
        dist = (sumf - 2.0 * s) + e2_j
        minv = jnp.min(dist, axis=1, keepdims=True)
        cand = jnp.where(dist == minv, iota, TCH)
        lidx = jnp.min(cand, axis=1, keepdims=True) + (c * TCB + j * TCH)
        if j == 0:
            run_val, run_idx = minv, lidx
        else:
            upd = minv < run_val
            run_val = jnp.where(upd, minv, run_val)
            run_idx = jnp.where(upd, lidx, run_idx)

    rows = pl.ds(r * TR, TR)

    @pl.when(c == 0)
    def _():
        bv_ref[rows, :] = run_val
        bi_ref[rows, :] = run_idx

    @pl.when(c > 0)
    def _():
        pv = bv_ref[rows, :]
        upd = run_val < pv
        best_v = jnp.where(upd, run_val, pv)
        best_i = jnp.where(upd, run_idx, bi_ref[rows, :])
        bv_ref[rows, :] = best_v
        bi_ref[rows, :] = best_i

    @pl.when(c == num_c - 1)
    def _():
        ind_ref[0, 0, :] = bi_ref[rows, 0]
        tile_sum = jnp.sum(bv_ref[rows, :])[None, None]
        sum_ref[...] = jnp.where(r == 0, tile_sum, sum_ref[...] + tile_sum)


def _make_sc_gather(n_rows, d):
    info = plsc.get_sparse_core_info()
    nw = info.num_cores * info.num_subcores
    assert n_rows % nw == 0
    b_per_w = n_rows // nw
    mesh = plsc.VectorSubcoreMesh(core_axis_name="c", subcore_axis_name="s")

    @functools.partial(
        pl.kernel,
        mesh=mesh,
        out_type=jax.ShapeDtypeStruct((n_rows, d), jnp.float32),
        scratch_types=[
            pltpu.VMEM((b_per_w,), jnp.int32),
            pltpu.VMEM((b_per_w, d), jnp.float32),
            pltpu.SemaphoreType.DMA,
        ],
    )
    def gather_k(table_hbm, idx_hbm, out_hbm, idx_v, rows_v, sem):
        wid = lax.axis_index("s") * info.num_cores + lax.axis_index("c")
        base = wid * b_per_w
        pltpu.sync_copy(idx_hbm.at[pl.ds(base, b_per_w)], idx_v)
        pltpu.async_copy(table_hbm.at[idx_v], rows_v, sem).wait()
        pltpu.sync_copy(rows_v, out_hbm.at[pl.ds(base, b_per_w)])

    return gather_k


def kernel(z, embed_weight):
    B, H, ch = z.shape
    n = B * H

    flatten = z.reshape(n, ch)

    ind3, dist_sum = pl.pallas_call(
        _argmin_body,
        grid=(C, n // TR),
        in_specs=[
            pl.BlockSpec((TR, ch), lambda c, r: (r, 0)),
            pl.BlockSpec((TCB, ch), lambda c, r: (c, 0)),
        ],
        out_specs=[
            pl.BlockSpec((1, 1, TR), lambda c, r: (r, 0, 0)),
            pl.BlockSpec((1, 1), lambda c, r: (0, 0)),
        ],
        out_shape=[
            jax.ShapeDtypeStruct((n // TR, 1, TR), jnp.int32),
            jax.ShapeDtypeStruct((1, 1), jnp.float32),
        ],
        scratch_shapes=[
            pltpu.VMEM((n, 1), jnp.float32),
            pltpu.VMEM((n, 1), jnp.int32),
            pltpu.VMEM((1, TCB), jnp.float32),
        ],
    )(flatten, embed_weight)

    ind_flat = ind3.reshape(n)
    gather = _make_sc_gather(n, ch)
    z_q = gather(embed_weight, ind_flat).reshape(B, H, ch)

    diff = dist_sum[0, 0] * (12.5 / (n * ch))
    ind = ind_flat.reshape(B, H)
    return z_q, diff, ind


# speedup vs baseline: 1.3252x; 1.0351x over previous
import functools

import jax
import jax.numpy as jnp
from jax import lax
from jax.experimental import pallas as pl
from jax.experimental.pallas import tpu as pltpu
from jax.experimental.pallas import tpu_sc as plsc

NUM_HIDDENS = 768
N_EMBED = 8192

TR = 512
TCB = 2048
C = N_EMBED // TCB
UNROLL = 4
TCH = TCB // UNROLL


def _argmin_body(f_ref, e_ref, ind_ref, sum_ref, bv_ref, bi_ref, e2_ref):
    c = pl.program_id(0)
    r = pl.program_id(1)
    num_c = pl.num_programs(0)

    f = f_ref[...]

    @pl.when(r == 0)
    def _():
        e = e_ref[...]
        ones = jnp.ones((1, e.shape[1]), jnp.float32)
        e2_ref[...] = lax.dot_general(ones, e * e, (((1,), (1,)), ((), ())),
                                      preferred_element_type=jnp.float32)

    sumf = jnp.sum(f * f, axis=1, keepdims=True)
    iota = lax.broadcasted_iota(jnp.int32, (TR, TCH), 1).astype(jnp.float32)

    run_val = None
    run_idx = None
    for j in range(UNROLL):
        e_j = e_ref[pl.ds(j * TCH, TCH), :]
        s = lax.dot_general(f, e_j, (((1,), (1,)), ((), ())),
                            preferred_element_type=jnp.float32)
        e2_j = e2_ref[0, pl.ds(j * TCH, TCH)][None, :]
        dist = (sumf - 2.0 * s) + e2_j
        minv = jnp.min(dist, axis=1, keepdims=True)
        cand = jnp.where(dist == minv, iota, jnp.float32(TCH))
        base_j = (c * TCB + j * TCH).astype(jnp.float32)
        lidx = jnp.min(cand, axis=1, keepdims=True) + base_j
        if j == 0:
            run_val, run_idx = minv, lidx
        else:
            upd = minv < run_val
            run_val = jnp.where(upd, minv, run_val)
            run_idx = jnp.where(upd, lidx, run_idx)

    rows = pl.ds(r * TR, TR)

    @pl.when(c == 0)
    def _():
        bv_ref[rows, :] = run_val
        bi_ref[rows, :] = run_idx

    @pl.when(c > 0)
    def _():
        pv = bv_ref[rows, :]
        upd = run_val < pv
        best_v = jnp.where(upd, run_val, pv)
        best_i = jnp.where(upd, run_idx, bi_ref[rows, :])
        bv_ref[rows, :] = best_v
        bi_ref[rows, :] = best_i

    @pl.when(c == num_c - 1)
    def _():
        ind_ref[0, 0, :] = bi_ref[rows, 0].astype(jnp.int32)
        tile_sum = jnp.sum(bv_ref[rows, :])[None, None]
        sum_ref[...] = jnp.where(r == 0, tile_sum, sum_ref[...] + tile_sum)


def _make_sc_gather(n_rows, d):
    info = plsc.get_sparse_core_info()
    nw = info.num_cores * info.num_subcores
    assert n_rows % nw == 0
    b_per_w = n_rows // nw
    mesh = plsc.VectorSubcoreMesh(core_axis_name="c", subcore_axis_name="s")

    @functools.partial(
        pl.kernel,
        mesh=mesh,
        out_type=jax.ShapeDtypeStruct((n_rows, d), jnp.float32),
        scratch_types=[
            pltpu.VMEM((b_per_w,), jnp.int32),
            pltpu.VMEM((b_per_w, d), jnp.float32),
            pltpu.SemaphoreType.DMA,
        ],
    )
    def gather_k(table_hbm, idx_hbm, out_hbm, idx_v, rows_v, sem):
        wid = lax.axis_index("s") * info.num_cores + lax.axis_index("c")
        base = wid * b_per_w
        pltpu.sync_copy(idx_hbm.at[pl.ds(base, b_per_w)], idx_v)
        pltpu.async_copy(table_hbm.at[idx_v], rows_v, sem).wait()
        pltpu.sync_copy(rows_v, out_hbm.at[pl.ds(base, b_per_w)])

    return gather_k


def kernel(z, embed_weight):
    B, H, ch = z.shape
    n = B * H

    flatten = z.reshape(n, ch)

    ind3, dist_sum = pl.pallas_call(
        _argmin_body,
        grid=(C, n // TR),
        in_specs=[
            pl.BlockSpec((TR, ch), lambda c, r: (r, 0)),
            pl.BlockSpec((TCB, ch), lambda c, r: (c, 0)),
        ],
        out_specs=[
            pl.BlockSpec((1, 1, TR), lambda c, r: (r, 0, 0)),
            pl.BlockSpec((1, 1), lambda c, r: (0, 0)),
        ],
        out_shape=[
            jax.ShapeDtypeStruct((n // TR, 1, TR), jnp.int32),
            jax.ShapeDtypeStruct((1, 1), jnp.float32),
        ],
        scratch_shapes=[
            pltpu.VMEM((n, 1), jnp.float32),
            pltpu.VMEM((n, 1), jnp.float32),
            pltpu.VMEM((1, TCB), jnp.float32),
        ],
    )(flatten, embed_weight)

    ind_flat = ind3.reshape(n)
    gather = _make_sc_gather(n, ch)
    z_q = gather(embed_weight, ind_flat).reshape(B, H, ch)

    diff = dist_sum[0, 0] * (12.5 / (n * ch))
    ind = ind_flat.reshape(B, H)
    return z_q, diff, ind


# speedup vs baseline: 1.4295x; 1.0787x over previous
import functools

import jax
import jax.numpy as jnp
from jax import lax
from jax.experimental import pallas as pl
from jax.experimental.pallas import tpu as pltpu
from jax.experimental.pallas import tpu_sc as plsc

NUM_HIDDENS = 768
N_EMBED = 8192

TR = 768
TCH = 512
UNROLL = N_EMBED // TCH


def _argmin_body(f_ref, e_ref, ind_ref, sum_ref, e2_ref):
    r = pl.program_id(0)

    f = f_ref[...]

    @pl.when(r == 0)
    def _():
        e = e_ref[...]
        ones = jnp.ones((1, e.shape[1]), jnp.float32)
        e2_ref[...] = lax.dot_general(ones, e * e, (((1,), (1,)), ((), ())),
                                      preferred_element_type=jnp.float32)

    sumf = jnp.sum(f * f, axis=1, keepdims=True)
    iota = lax.broadcasted_iota(jnp.int32, (TR, TCH), 1).astype(jnp.float32)

    run_val = None
    run_idx = None
    for j in range(UNROLL):
        e_j = e_ref[pl.ds(j * TCH, TCH), :]
        s = lax.dot_general(f, e_j, (((1,), (1,)), ((), ())),
                            preferred_element_type=jnp.float32)
        e2_j = e2_ref[0, pl.ds(j * TCH, TCH)][None, :]
        dist = (sumf - 2.0 * s) + e2_j
        minv = jnp.min(dist, axis=1, keepdims=True)
        cand = jnp.where(dist == minv, iota, jnp.float32(TCH))
        lidx = jnp.min(cand, axis=1, keepdims=True) + jnp.float32(j * TCH)
        if j == 0:
            run_val, run_idx = minv, lidx
        else:
            upd = minv < run_val
            run_val = jnp.where(upd, minv, run_val)
            run_idx = jnp.where(upd, lidx, run_idx)

    ind_ref[0, 0, :] = run_idx[:, 0].astype(jnp.int32)
    tile_sum = jnp.sum(run_val)[None, None]
    sum_ref[...] = jnp.where(r == 0, tile_sum, sum_ref[...] + tile_sum)


def _make_sc_gather(n_rows, d):
    info = plsc.get_sparse_core_info()
    nw = info.num_cores * info.num_subcores
    assert n_rows % nw == 0
    b_per_w = n_rows // nw
    mesh = plsc.VectorSubcoreMesh(core_axis_name="c", subcore_axis_name="s")

    @functools.partial(
        pl.kernel,
        mesh=mesh,
        out_type=jax.ShapeDtypeStruct((n_rows, d), jnp.float32),
        scratch_types=[
            pltpu.VMEM((b_per_w,), jnp.int32),
            pltpu.VMEM((b_per_w, d), jnp.float32),
            pltpu.SemaphoreType.DMA,
        ],
    )
    def gather_k(table_hbm, idx_hbm, out_hbm, idx_v, rows_v, sem):
        wid = lax.axis_index("s") * info.num_cores + lax.axis_index("c")
        base = wid * b_per_w
        pltpu.sync_copy(idx_hbm.at[pl.ds(base, b_per_w)], idx_v)
        pltpu.async_copy(table_hbm.at[idx_v], rows_v, sem).wait()
        pltpu.sync_copy(rows_v, out_hbm.at[pl.ds(base, b_per_w)])

    return gather_k


def kernel(z, embed_weight):
    B, H, ch = z.shape
    n = B * H

    flatten = z.reshape(n, ch)

    ind3, dist_sum = pl.pallas_call(
        _argmin_body,
        grid=(n // TR,),
        in_specs=[
            pl.BlockSpec((TR, ch), lambda r: (r, 0)),
            pl.BlockSpec((N_EMBED, ch), lambda r: (0, 0)),
        ],
        out_specs=[
            pl.BlockSpec((1, 1, TR), lambda r: (r, 0, 0)),
            pl.BlockSpec((1, 1), lambda r: (0, 0)),
        ],
        out_shape=[
            jax.ShapeDtypeStruct((n // TR, 1, TR), jnp.int32),
            jax.ShapeDtypeStruct((1, 1), jnp.float32),
        ],
        scratch_shapes=[
            pltpu.VMEM((1, N_EMBED), jnp.float32),
        ],
    )(flatten, embed_weight)

    ind_flat = ind3.reshape(n)
    gather = _make_sc_gather(n, ch)
    z_q = gather(embed_weight, ind_flat).reshape(B, H, ch)

    diff = dist_sum[0, 0] * (12.5 / (n * ch))
    ind = ind_flat.reshape(B, H)
    return z_q, diff, ind
